# baseline (speedup 1.0000x reference)
import functools

import jax
import jax.numpy as jnp
from jax import lax
from jax.experimental import pallas as pl
from jax.experimental.pallas import tpu as pltpu
from jax.experimental.pallas import tpu_sc as plsc

_G = 32


@functools.lru_cache(maxsize=None)
def _make_gather_kernel(B, D, NC, NS):
    NW = NC * NS
    b_per_w = B // NW
    n_rounds = b_per_w // _G
    mesh = plsc.VectorSubcoreMesh(core_axis_name="c", subcore_axis_name="s")

    @functools.partial(
        pl.kernel,
        mesh=mesh,
        out_type=(
            jax.ShapeDtypeStruct((B, D), jnp.float32),
            jax.ShapeDtypeStruct((B, D), jnp.float32),
        ),
        scratch_types=[
            pltpu.VMEM((b_per_w,), jnp.int32),
            pltpu.VMEM((b_per_w,), jnp.int32),
            pltpu.VMEM((_G, 8, D), jnp.float32),
            pltpu.VMEM((_G, 8, D), jnp.float32),
            pltpu.VMEM((_G, D), jnp.float32),
            pltpu.VMEM((_G, D), jnp.float32),
            pltpu.SemaphoreType.DMA,
            pltpu.SemaphoreType.DMA,
        ],
    )
    def gather_scale(users_hbm, items_hbm, ut_hbm, it_hbm,
                     out_u_hbm, out_i_hbm,
                     uidx_v, iidx_v, tiles_u, tiles_i, orow_u, orow_i,
                     sem_u, sem_i):
        wid = lax.axis_index("s") * NC + lax.axis_index("c")
        base = wid * b_per_w
        pltpu.sync_copy(users_hbm.at[pl.ds(base, b_per_w)], uidx_v)
        pltpu.sync_copy(items_hbm.at[pl.ds(base, b_per_w)], iidx_v)

        def round_body(g, carry):
            uvs = [uidx_v[pl.ds(g * _G + h * 16, 16)] for h in range(_G // 16)]
            ivs = [iidx_v[pl.ds(g * _G + h * 16, 16)] for h in range(_G // 16)]
            ucopies, icopies = [], []
            for h, (uv, iv) in enumerate(zip(uvs, ivs)):
                ub = (uv >> 3) << 3
                ib = (iv >> 3) << 3
                for s in range(16):
                    uo = pl.multiple_of(ub[s], 8)
                    io = pl.multiple_of(ib[s], 8)
                    ucopies.append(pltpu.async_copy(
                        ut_hbm.at[pl.ds(uo, 8)],
                        tiles_u.at[h * 16 + s], sem_u))
                    icopies.append(pltpu.async_copy(
                        it_hbm.at[pl.ds(io, 8)],
                        tiles_i.at[h * 16 + s], sem_i))
            for c in ucopies:
                c.wait()
            for h, uv in enumerate(uvs):
                for s in range(16):
                    r = h * 16 + s
                    sub = uv[s] & 7
                    for k in range(D // 16):
                        sl = pl.ds(k * 16, 16)
                        orow_u[r, sl] = tiles_u[r, sub, sl] * 0.25
            pltpu.sync_copy(orow_u, out_u_hbm.at[pl.ds(base + g * _G, _G)])
            for c in icopies:
                c.wait()
            for h, iv in enumerate(ivs):
                for s in range(16):
                    r = h * 16 + s
                    sub = iv[s] & 7
                    for k in range(D // 16):
                        sl = pl.ds(k * 16, 16)
                        orow_i[r, sl] = tiles_i[r, sub, sl] * 0.25
            pltpu.sync_copy(orow_i, out_i_hbm.at[pl.ds(base + g * _G, _G)])
            return carry

        lax.fori_loop(0, n_rounds, round_body, 0)

    return gather_scale


def kernel(users, items, user_table, item_table, edge_user, edge_item):
    B = users.shape[0]
    D = user_table.shape[1]
    info = plsc.get_sparse_core_info()
    fn = _make_gather_kernel(B, D, info.num_cores, info.num_subcores)
    return fn(users, items, user_table, item_table)


# speedup vs baseline: 1.2107x; 1.2107x over previous
import functools

import jax
import jax.numpy as jnp
from jax import lax
from jax.experimental import pallas as pl
from jax.experimental.pallas import tpu as pltpu
from jax.experimental.pallas import tpu_sc as plsc

_CHUNK = 128
_DP = 128


@functools.lru_cache(maxsize=None)
def _make_gather_kernel(B, D, NC, NS):
    NW = NC * NS
    b_per_w = B // NW
    n_chunks = b_per_w // _CHUNK
    mesh = plsc.VectorSubcoreMesh(core_axis_name="c", subcore_axis_name="s")

    @functools.partial(
        pl.kernel,
        mesh=mesh,
        out_type=(
            jax.ShapeDtypeStruct((B, _DP), jnp.float32),
            jax.ShapeDtypeStruct((B, _DP), jnp.float32),
        ),
        scratch_types=[
            pltpu.VMEM((n_chunks, _CHUNK), jnp.int32),
            pltpu.VMEM((n_chunks, _CHUNK), jnp.int32),
            pltpu.VMEM((b_per_w, _DP), jnp.float32),
            pltpu.SemaphoreType.DMA,
        ],
    )
    def gather_scale(users_hbm, items_hbm, ut_hbm, it_hbm,
                     out_u_hbm, out_i_hbm,
                     uidx_v, iidx_v, rows_v, sem):
        wid = lax.axis_index("s") * NC + lax.axis_index("c")
        base = wid * b_per_w
        for j in range(n_chunks):
            pltpu.sync_copy(users_hbm.at[pl.ds(base + j * _CHUNK, _CHUNK)],
                            uidx_v.at[j])
            pltpu.sync_copy(items_hbm.at[pl.ds(base + j * _CHUNK, _CHUNK)],
                            iidx_v.at[j])

        def run_phase(idx_v, tab_hbm, out_hbm):
            copies = [
                pltpu.async_copy(
                    tab_hbm.at[idx_v.at[j]],
                    rows_v.at[pl.ds(j * _CHUNK, _CHUNK)], sem)
                for j in range(n_chunks)
            ]
            for c in copies:
                c.wait()

            def scale_row(r, carry):
                for k in range(D // 16):
                    sl = pl.ds(k * 16, 16)
                    rows_v[r, sl] = rows_v[r, sl] * 0.25
                return carry

            lax.fori_loop(0, b_per_w, scale_row, 0)
            pltpu.sync_copy(rows_v, out_hbm.at[pl.ds(base, b_per_w)])

        run_phase(uidx_v, ut_hbm, out_u_hbm)
        run_phase(iidx_v, it_hbm, out_i_hbm)

    return gather_scale


def kernel(users, items, user_table, item_table, edge_user, edge_item):
    B = users.shape[0]
    D = user_table.shape[1]
    info = plsc.get_sparse_core_info()
    fn = _make_gather_kernel(B, D, info.num_cores, info.num_subcores)
    utp = jnp.pad(user_table, ((0, 0), (0, _DP - D)))
    itp = jnp.pad(item_table, ((0, 0), (0, _DP - D)))
    out_u, out_i = fn(users, items, utp, itp)
    return out_u[:, :D], out_i[:, :D]
